# baseline (speedup 1.0000x reference)
import functools

import jax
import jax.numpy as jnp
from jax import lax
from jax.experimental import pallas as pl
from jax.experimental.pallas import tpu as pltpu
from jax.experimental.pallas import tpu_sc as plsc

N = 10000
E = 320000
D = 128
NG = 16

NC = 2
NS = 16
NW = NC * NS
EPW = E // NW
CHUNK = 80
NCH = EPW // CHUNK
NPAD = 10240
RPS = NPAD // NS
ZR = 8
DPW = NPAD // NS


def _sc_mesh():
    return plsc.VectorSubcoreMesh(core_axis_name="c", subcore_axis_name="s")


def _deg_body(epk_hbm, out_hbm, pidx_v, didx_v, ones_v, zb_v, acc):
    c = lax.axis_index("c")
    s = lax.axis_index("s")
    wid = s * NC + c

    z16 = jnp.zeros((16,), jnp.float32)
    o16 = jnp.ones((16,), jnp.float32)

    def fill(i, _):
        zb_v[pl.ds(i * 16, 16)] = z16
        return 0

    lax.fori_loop(0, DPW // 16, fill, 0, unroll=8)
    for i in range(CHUNK // 16):
        ones_v[pl.ds(i * 16, 16)] = o16
    pltpu.sync_copy(zb_v, acc.at[pl.ds(s * DPW, DPW)])
    pltpu.sync_copy(epk_hbm.at[wid], pidx_v)
    plsc.subcore_barrier()

    def step(k, _):
        for i in range(CHUNK // 16):
            p = pidx_v[k, pl.ds(i * 16, 16)]
            didx_v[pl.ds(i * 16, 16)] = lax.shift_right_logical(p, 16)
        pltpu.sync_copy(ones_v, acc.at[didx_v], add=True)
        return 0

    lax.fori_loop(0, NCH, step, 0)
    plsc.subcore_barrier()
    pltpu.sync_copy(acc.at[pl.ds(s * DPW, DPW)], out_hbm.at[c, pl.ds(s * DPW, DPW)])


def _deg_call(epk):
    return pl.kernel(
        _deg_body,
        out_type=jax.ShapeDtypeStruct((NC, NPAD), jnp.float32),
        mesh=_sc_mesh(),
        scratch_types=[
            pltpu.VMEM((NCH, CHUNK), jnp.int32),
            pltpu.VMEM((CHUNK,), jnp.int32),
            pltpu.VMEM((CHUNK,), jnp.float32),
            pltpu.VMEM((DPW,), jnp.float32),
            pltpu.VMEM_SHARED((NPAD,), jnp.float32),
        ],
    )(epk)


def _edge_body(epk_hbm, y_hbm, out_hbm, pidx_v, sidx_v, didx_v, rows_v, zb_v, acc,
               sem0, sem1, isem):
    c = lax.axis_index("c")
    s = lax.axis_index("s")
    wid = s * NC + c

    icp = pltpu.make_async_copy(epk_hbm.at[wid], pidx_v, isem)
    icp.start()

    z16 = jnp.zeros((16,), jnp.float32)

    def fill(i, _):
        zb_v[i // 8, pl.ds((i % 8) * 16, 16)] = z16
        return 0

    lax.fori_loop(0, ZR * (D // 16), fill, 0, unroll=8)

    def zero(j, _):
        pltpu.sync_copy(zb_v, acc.at[pl.ds(s * RPS + j * ZR, ZR), :])
        return 0

    lax.fori_loop(0, RPS // ZR, zero, 0)
    icp.wait()
    plsc.subcore_barrier()

    def unpack(k, buf):
        for i in range(CHUNK // 16):
            p = pidx_v[k, pl.ds(i * 16, 16)]
            sidx_v[buf, pl.ds(i * 16, 16)] = lax.bitwise_and(p, 0xFFFF)
            didx_v[buf, pl.ds(i * 16, 16)] = lax.shift_right_logical(p, 16)

    def g_start(buf, sem):
        pltpu.make_async_copy(y_hbm.at[sidx_v.at[buf]], rows_v.at[buf], sem).start()

    def g_wait(buf, sem):
        pltpu.make_async_copy(y_hbm.at[sidx_v.at[buf]], rows_v.at[buf], sem).wait()

    def sc_add(buf):
        pltpu.sync_copy(rows_v.at[buf], acc.at[didx_v.at[buf]], add=True)

    unpack(0, 0)
    g_start(0, sem0)

    def pair(kk, _):
        k = kk * 2
        unpack(k + 1, 1)
        g_start(1, sem1)
        g_wait(0, sem0)
        sc_add(0)
        unpack(k + 2, 0)
        g_start(0, sem0)
        g_wait(1, sem1)
        sc_add(1)
        return 0

    lax.fori_loop(0, (NCH - 1) // 2, pair, 0)
    g_wait(0, sem0)
    sc_add(0)

    plsc.subcore_barrier()
    pltpu.sync_copy(
        acc.at[pl.ds(s * RPS, RPS), :], out_hbm.at[c, pl.ds(s * RPS, RPS), :]
    )


def _edge_call(epk, y):
    return pl.kernel(
        _edge_body,
        out_type=jax.ShapeDtypeStruct((NC, NPAD, D), jnp.float32),
        mesh=_sc_mesh(),
        scratch_types=[
            pltpu.VMEM((NCH, CHUNK), jnp.int32),
            pltpu.VMEM((2, CHUNK), jnp.int32),
            pltpu.VMEM((2, CHUNK), jnp.int32),
            pltpu.VMEM((2, CHUNK, D), jnp.float32),
            pltpu.VMEM((ZR, D), jnp.float32),
            pltpu.VMEM_SHARED((NPAD, D), jnp.float32),
            pltpu.SemaphoreType.DMA,
            pltpu.SemaphoreType.DMA,
            pltpu.SemaphoreType.DMA,
        ],
    )(epk, y)


_BR = 1000
_GRID = N // _BR


def _tc_xw_body(x_ref, w_ref, o_ref):
    o_ref[...] = jnp.dot(x_ref[...], w_ref[...], preferred_element_type=jnp.float32)


def _tc_xw(x, w1):
    return pl.pallas_call(
        _tc_xw_body,
        grid=(_GRID,),
        in_specs=[
            pl.BlockSpec((_BR, D), lambda i: (i, 0)),
            pl.BlockSpec((D, D), lambda i: (0, 0)),
        ],
        out_specs=pl.BlockSpec((_BR, D), lambda i: (i, 0)),
        out_shape=jax.ShapeDtypeStruct((N, D), jnp.float32),
    )(x, w1)


def _tc_scale_body(xw_ref, d0_ref, d1_ref, y_ref, dinv_ref):
    deg = d0_ref[...] + d1_ref[...] + 1.0
    dinv = lax.rsqrt(deg)
    y_ref[...] = xw_ref[...] * dinv
    dinv_ref[...] = dinv


def _tc_scale(xw, deg0, deg1):
    return pl.pallas_call(
        _tc_scale_body,
        grid=(_GRID,),
        in_specs=[
            pl.BlockSpec((_BR, D), lambda i: (i, 0)),
            pl.BlockSpec((_BR, 1), lambda i: (i, 0)),
            pl.BlockSpec((_BR, 1), lambda i: (i, 0)),
        ],
        out_specs=[
            pl.BlockSpec((_BR, D), lambda i: (i, 0)),
            pl.BlockSpec((_BR, 1), lambda i: (i, 0)),
        ],
        out_shape=[
            jax.ShapeDtypeStruct((N, D), jnp.float32),
            jax.ShapeDtypeStruct((N, 1), jnp.float32),
        ],
    )(xw, deg0, deg1)


def _ln(h, g, b, eps=1e-5):
    mu = jnp.mean(h, axis=-1, keepdims=True)
    var = jnp.mean((h - mu) ** 2, axis=-1, keepdims=True)
    return (h - mu) * lax.rsqrt(var + eps) * g + b


def _tc_mid_body(s0_ref, s1_ref, y_ref, dinv_ref, b_ref, g_ref, bl_ref, w_ref, o_ref):
    dinv = dinv_ref[...]
    conv = dinv * (s0_ref[...] + s1_ref[...] + y_ref[...]) + b_ref[...]
    h = _ln(jnp.maximum(conv, 0.0), g_ref[...], bl_ref[...])
    xw = jnp.dot(h, w_ref[...], preferred_element_type=jnp.float32)
    o_ref[...] = xw * dinv


def _tc_mid(s0, s1, y, dinv, b, g, bl, w_next):
    row = lambda i: (i, 0)
    return pl.pallas_call(
        _tc_mid_body,
        grid=(_GRID,),
        in_specs=[
            pl.BlockSpec((_BR, D), row),
            pl.BlockSpec((_BR, D), row),
            pl.BlockSpec((_BR, D), row),
            pl.BlockSpec((_BR, 1), row),
            pl.BlockSpec((1, D), lambda i: (0, 0)),
            pl.BlockSpec((1, D), lambda i: (0, 0)),
            pl.BlockSpec((1, D), lambda i: (0, 0)),
            pl.BlockSpec((D, D), lambda i: (0, 0)),
        ],
        out_specs=pl.BlockSpec((_BR, D), row),
        out_shape=jax.ShapeDtypeStruct((N, D), jnp.float32),
    )(s0, s1, y, dinv, b, g, bl, w_next)


def _tc_last_body(
    s0_ref, s1_ref, y_ref, dinv_ref, b_ref, g_ref, bl_ref, batch_ref,
    wp1_ref, bp1_ref, wp2_ref, bp2_ref, wp3_ref, bp3_ref,
    emb_ref, logits_ref, probs_ref,
):
    dinv = dinv_ref[...]
    emb = dinv * (s0_ref[...] + s1_ref[...] + y_ref[...]) + b_ref[...]
    emb_ref[...] = emb
    h = _ln(jnp.maximum(emb, 0.0), g_ref[...], bl_ref[...])

    gids = lax.broadcasted_iota(jnp.int32, (NG, N), 0)
    onehot = (gids == batch_ref[...]).astype(jnp.float32)
    sums = jnp.dot(onehot, h, preferred_element_type=jnp.float32)
    cnt = jnp.sum(onehot, axis=1, keepdims=True)
    pooled = sums / jnp.maximum(cnt, 1.0)

    z = jnp.dot(pooled, wp1_ref[...], preferred_element_type=jnp.float32) + bp1_ref[...]
    z = jnp.dot(z, wp2_ref[...], preferred_element_type=jnp.float32) + bp2_ref[...]
    logits = jnp.dot(z, wp3_ref[...], preferred_element_type=jnp.float32) + bp3_ref[...]
    logits_ref[...] = logits
    m = jnp.max(logits, axis=1, keepdims=True)
    lse = jnp.log(jnp.sum(jnp.exp(logits - m), axis=1, keepdims=True)) + m
    probs_ref[...] = logits - lse


def _tc_last(s0, s1, y, dinv, b, g, bl, batch_t, wp1, bp1, wp2, bp2, wp3, bp3):
    return pl.pallas_call(
        _tc_last_body,
        out_shape=[
            jax.ShapeDtypeStruct((N, D), jnp.float32),
            jax.ShapeDtypeStruct((NG, NG), jnp.float32),
            jax.ShapeDtypeStruct((NG, NG), jnp.float32),
        ],
    )(s0, s1, y, dinv, b, g, bl, batch_t, wp1, bp1, wp2, bp2, wp3, bp3)


def kernel(x, edge_index, batch, W1, b1, ln1_g, ln1_b, convs_W, convs_b,
           lns_g, lns_b, Wp1, bp1, Wp2, bp2, Wp3, bp3):
    epk = (edge_index[0] | (edge_index[1] << 16)).reshape(NW, NCH, CHUNK)

    deg = _deg_call(epk)
    deg0 = deg[0, :N].reshape(N, 1)
    deg1 = deg[1, :N].reshape(N, 1)

    xw1 = _tc_xw(x, W1)
    y, dinv = _tc_scale(xw1, deg0, deg1)

    biases = [b1] + list(convs_b)
    gammas = [ln1_g] + list(lns_g)
    betas = [ln1_b] + list(lns_b)
    ws = list(convs_W)

    for i in range(3):
        s = _edge_call(epk, y)
        y = _tc_mid(
            s[0], s[1], y, dinv,
            biases[i].reshape(1, D), gammas[i].reshape(1, D),
            betas[i].reshape(1, D), ws[i],
        )

    s = _edge_call(epk, y)
    emb, logits, probs = _tc_last(
        s[0, :N], s[1, :N], y, dinv,
        biases[3].reshape(1, D), gammas[3].reshape(1, D), betas[3].reshape(1, D),
        batch.reshape(1, N),
        Wp1, bp1.reshape(1, -1), Wp2, bp2.reshape(1, -1), Wp3, bp3.reshape(1, -1),
    )
    return emb, logits, probs


# speedup vs baseline: 1.0121x; 1.0121x over previous
import functools

import jax
import jax.numpy as jnp
from jax import lax
from jax.experimental import pallas as pl
from jax.experimental.pallas import tpu as pltpu
from jax.experimental.pallas import tpu_sc as plsc

N = 10000
E = 320000
D = 128
NG = 16

NC = 2
NS = 16
NW = NC * NS
EPW = E // NW
CHUNK = 80
NCH = EPW // CHUNK
NPAD = 10240
RPS = NPAD // NS
ZR = 8
DPW = NPAD // NS


def _sc_mesh():
    return plsc.VectorSubcoreMesh(core_axis_name="c", subcore_axis_name="s")


def _deg_body(epk_hbm, out_hbm, pidx_v, didx_v, ones_v, zb_v, acc, dsem0, dsem1):
    c = lax.axis_index("c")
    s = lax.axis_index("s")
    wid = s * NC + c

    z16 = jnp.zeros((16,), jnp.float32)
    o16 = jnp.ones((16,), jnp.float32)

    def fill(i, _):
        zb_v[pl.ds(i * 16, 16)] = z16
        return 0

    lax.fori_loop(0, DPW // 16, fill, 0, unroll=8)
    for i in range(CHUNK // 16):
        ones_v[pl.ds(i * 16, 16)] = o16
    pltpu.sync_copy(zb_v, acc.at[pl.ds(s * DPW, DPW)])
    pltpu.sync_copy(epk_hbm.at[wid], pidx_v)
    plsc.subcore_barrier()

    def unpack(k, buf):
        for i in range(CHUNK // 16):
            p = pidx_v[k, pl.ds(i * 16, 16)]
            didx_v[buf, pl.ds(i * 16, 16)] = lax.shift_right_logical(p, 16)

    def s_start(buf, sem):
        pltpu.make_async_copy(ones_v, acc.at[didx_v.at[buf]], sem).start(add=True)

    def s_wait(buf, sem):
        pltpu.make_async_copy(ones_v, acc.at[didx_v.at[buf]], sem).wait()

    unpack(0, 0)
    s_start(0, dsem0)

    def pair(kk, _):
        k = kk * 2
        unpack(k + 1, 1)
        s_start(1, dsem1)
        s_wait(0, dsem0)
        unpack(k + 2, 0)
        s_start(0, dsem0)
        s_wait(1, dsem1)
        return 0

    lax.fori_loop(0, (NCH - 1) // 2, pair, 0)
    s_wait(0, dsem0)
    plsc.subcore_barrier()
    pltpu.sync_copy(acc.at[pl.ds(s * DPW, DPW)], out_hbm.at[c, pl.ds(s * DPW, DPW)])


def _deg_call(epk):
    return pl.kernel(
        _deg_body,
        out_type=jax.ShapeDtypeStruct((NC, NPAD), jnp.float32),
        mesh=_sc_mesh(),
        scratch_types=[
            pltpu.VMEM((NCH, CHUNK), jnp.int32),
            pltpu.VMEM((2, CHUNK), jnp.int32),
            pltpu.VMEM((CHUNK,), jnp.float32),
            pltpu.VMEM((DPW,), jnp.float32),
            pltpu.VMEM_SHARED((NPAD,), jnp.float32),
            pltpu.SemaphoreType.DMA,
            pltpu.SemaphoreType.DMA,
        ],
    )(epk)


def _edge_body(epk_hbm, y_hbm, out_hbm, pidx_v, sidx_v, didx_v, rows_v, zb_v, acc,
               sem0, sem1, isem):
    c = lax.axis_index("c")
    s = lax.axis_index("s")
    wid = s * NC + c

    icp = pltpu.make_async_copy(epk_hbm.at[wid], pidx_v, isem)
    icp.start()

    z16 = jnp.zeros((16,), jnp.float32)

    def fill(i, _):
        zb_v[i // 8, pl.ds((i % 8) * 16, 16)] = z16
        return 0

    lax.fori_loop(0, ZR * (D // 16), fill, 0, unroll=8)

    def zero(j, _):
        pltpu.sync_copy(zb_v, acc.at[pl.ds(s * RPS + j * ZR, ZR), :])
        return 0

    lax.fori_loop(0, RPS // ZR, zero, 0)
    icp.wait()
    plsc.subcore_barrier()

    def unpack(k, buf):
        for i in range(CHUNK // 16):
            p = pidx_v[k, pl.ds(i * 16, 16)]
            sidx_v[buf, pl.ds(i * 16, 16)] = lax.bitwise_and(p, 0xFFFF)
            didx_v[buf, pl.ds(i * 16, 16)] = lax.shift_right_logical(p, 16)

    def g_start(buf, sem):
        pltpu.make_async_copy(y_hbm.at[sidx_v.at[buf]], rows_v.at[buf], sem).start()

    def g_wait(buf, sem):
        pltpu.make_async_copy(y_hbm.at[sidx_v.at[buf]], rows_v.at[buf], sem).wait()

    def sc_add(buf):
        pltpu.sync_copy(rows_v.at[buf], acc.at[didx_v.at[buf]], add=True)

    unpack(0, 0)
    g_start(0, sem0)

    def pair(kk, _):
        k = kk * 2
        unpack(k + 1, 1)
        g_start(1, sem1)
        g_wait(0, sem0)
        sc_add(0)
        unpack(k + 2, 0)
        g_start(0, sem0)
        g_wait(1, sem1)
        sc_add(1)
        return 0

    lax.fori_loop(0, (NCH - 1) // 2, pair, 0)
    g_wait(0, sem0)
    sc_add(0)

    plsc.subcore_barrier()
    pltpu.sync_copy(
        acc.at[pl.ds(s * RPS, RPS), :], out_hbm.at[c, pl.ds(s * RPS, RPS), :]
    )


def _edge_call(epk, y):
    return pl.kernel(
        _edge_body,
        out_type=jax.ShapeDtypeStruct((NC, NPAD, D), jnp.float32),
        mesh=_sc_mesh(),
        scratch_types=[
            pltpu.VMEM((NCH, CHUNK), jnp.int32),
            pltpu.VMEM((2, CHUNK), jnp.int32),
            pltpu.VMEM((2, CHUNK), jnp.int32),
            pltpu.VMEM((2, CHUNK, D), jnp.float32),
            pltpu.VMEM((ZR, D), jnp.float32),
            pltpu.VMEM_SHARED((NPAD, D), jnp.float32),
            pltpu.SemaphoreType.DMA,
            pltpu.SemaphoreType.DMA,
            pltpu.SemaphoreType.DMA,
        ],
    )(epk, y)


_BR = 1000
_GRID = N // _BR


def _tc_xw_body(x_ref, w_ref, o_ref):
    o_ref[...] = jnp.dot(x_ref[...], w_ref[...], preferred_element_type=jnp.float32)


def _tc_xw(x, w1):
    return pl.pallas_call(
        _tc_xw_body,
        grid=(_GRID,),
        in_specs=[
            pl.BlockSpec((_BR, D), lambda i: (i, 0)),
            pl.BlockSpec((D, D), lambda i: (0, 0)),
        ],
        out_specs=pl.BlockSpec((_BR, D), lambda i: (i, 0)),
        out_shape=jax.ShapeDtypeStruct((N, D), jnp.float32),
    )(x, w1)


def _tc_scale_body(xw_ref, d0_ref, d1_ref, y_ref, dinv_ref):
    deg = d0_ref[...] + d1_ref[...] + 1.0
    dinv = lax.rsqrt(deg)
    y_ref[...] = xw_ref[...] * dinv
    dinv_ref[...] = dinv


def _tc_scale(xw, deg0, deg1):
    return pl.pallas_call(
        _tc_scale_body,
        grid=(_GRID,),
        in_specs=[
            pl.BlockSpec((_BR, D), lambda i: (i, 0)),
            pl.BlockSpec((_BR, 1), lambda i: (i, 0)),
            pl.BlockSpec((_BR, 1), lambda i: (i, 0)),
        ],
        out_specs=[
            pl.BlockSpec((_BR, D), lambda i: (i, 0)),
            pl.BlockSpec((_BR, 1), lambda i: (i, 0)),
        ],
        out_shape=[
            jax.ShapeDtypeStruct((N, D), jnp.float32),
            jax.ShapeDtypeStruct((N, 1), jnp.float32),
        ],
    )(xw, deg0, deg1)


def _ln(h, g, b, eps=1e-5):
    mu = jnp.mean(h, axis=-1, keepdims=True)
    var = jnp.mean((h - mu) ** 2, axis=-1, keepdims=True)
    return (h - mu) * lax.rsqrt(var + eps) * g + b


def _tc_mid_body(s0_ref, s1_ref, y_ref, dinv_ref, b_ref, g_ref, bl_ref, w_ref, o_ref):
    dinv = dinv_ref[...]
    conv = dinv * (s0_ref[...] + s1_ref[...] + y_ref[...]) + b_ref[...]
    h = _ln(jnp.maximum(conv, 0.0), g_ref[...], bl_ref[...])
    xw = jnp.dot(h, w_ref[...], preferred_element_type=jnp.float32)
    o_ref[...] = xw * dinv


def _tc_mid(s0, s1, y, dinv, b, g, bl, w_next):
    row = lambda i: (i, 0)
    return pl.pallas_call(
        _tc_mid_body,
        grid=(_GRID,),
        in_specs=[
            pl.BlockSpec((_BR, D), row),
            pl.BlockSpec((_BR, D), row),
            pl.BlockSpec((_BR, D), row),
            pl.BlockSpec((_BR, 1), row),
            pl.BlockSpec((1, D), lambda i: (0, 0)),
            pl.BlockSpec((1, D), lambda i: (0, 0)),
            pl.BlockSpec((1, D), lambda i: (0, 0)),
            pl.BlockSpec((D, D), lambda i: (0, 0)),
        ],
        out_specs=pl.BlockSpec((_BR, D), row),
        out_shape=jax.ShapeDtypeStruct((N, D), jnp.float32),
    )(s0, s1, y, dinv, b, g, bl, w_next)


def _tc_last_body(
    s0_ref, s1_ref, y_ref, dinv_ref, b_ref, g_ref, bl_ref, batch_ref,
    wp1_ref, bp1_ref, wp2_ref, bp2_ref, wp3_ref, bp3_ref,
    emb_ref, logits_ref, probs_ref,
):
    dinv = dinv_ref[...]
    emb = dinv * (s0_ref[...] + s1_ref[...] + y_ref[...]) + b_ref[...]
    emb_ref[...] = emb
    h = _ln(jnp.maximum(emb, 0.0), g_ref[...], bl_ref[...])

    gids = lax.broadcasted_iota(jnp.int32, (NG, N), 0)
    onehot = (gids == batch_ref[...]).astype(jnp.float32)
    sums = jnp.dot(onehot, h, preferred_element_type=jnp.float32)
    cnt = jnp.sum(onehot, axis=1, keepdims=True)
    pooled = sums / jnp.maximum(cnt, 1.0)

    z = jnp.dot(pooled, wp1_ref[...], preferred_element_type=jnp.float32) + bp1_ref[...]
    z = jnp.dot(z, wp2_ref[...], preferred_element_type=jnp.float32) + bp2_ref[...]
    logits = jnp.dot(z, wp3_ref[...], preferred_element_type=jnp.float32) + bp3_ref[...]
    logits_ref[...] = logits
    m = jnp.max(logits, axis=1, keepdims=True)
    lse = jnp.log(jnp.sum(jnp.exp(logits - m), axis=1, keepdims=True)) + m
    probs_ref[...] = logits - lse


def _tc_last(s0, s1, y, dinv, b, g, bl, batch_t, wp1, bp1, wp2, bp2, wp3, bp3):
    z = lambda i: (0, 0)
    return pl.pallas_call(
        _tc_last_body,
        grid=(1,),
        in_specs=[
            pl.BlockSpec((N, D), z),
            pl.BlockSpec((N, D), z),
            pl.BlockSpec((N, D), z),
            pl.BlockSpec((N, 1), z),
            pl.BlockSpec((1, D), z),
            pl.BlockSpec((1, D), z),
            pl.BlockSpec((1, D), z),
            pl.BlockSpec((1, N), z),
            pl.BlockSpec((D, 2 * D), z),
            pl.BlockSpec((1, 2 * D), z),
            pl.BlockSpec((2 * D, D), z),
            pl.BlockSpec((1, D), z),
            pl.BlockSpec((D, NG), z),
            pl.BlockSpec((1, NG), z),
        ],
        out_specs=[
            pl.BlockSpec((N, D), z),
            pl.BlockSpec((NG, NG), z),
            pl.BlockSpec((NG, NG), z),
        ],
        out_shape=[
            jax.ShapeDtypeStruct((N, D), jnp.float32),
            jax.ShapeDtypeStruct((NG, NG), jnp.float32),
            jax.ShapeDtypeStruct((NG, NG), jnp.float32),
        ],
    )(s0, s1, y, dinv, b, g, bl, batch_t, wp1, bp1, wp2, bp2, wp3, bp3)


def kernel(x, edge_index, batch, W1, b1, ln1_g, ln1_b, convs_W, convs_b,
           lns_g, lns_b, Wp1, bp1, Wp2, bp2, Wp3, bp3):
    epk = (edge_index[0] | (edge_index[1] << 16)).reshape(NW, NCH, CHUNK)

    deg = _deg_call(epk)
    deg0 = deg[0, :N].reshape(N, 1)
    deg1 = deg[1, :N].reshape(N, 1)

    xw1 = _tc_xw(x, W1)
    y, dinv = _tc_scale(xw1, deg0, deg1)

    biases = [b1] + list(convs_b)
    gammas = [ln1_g] + list(lns_g)
    betas = [ln1_b] + list(lns_b)
    ws = list(convs_W)

    for i in range(3):
        s = _edge_call(epk, y)
        y = _tc_mid(
            s[0], s[1], y, dinv,
            biases[i].reshape(1, D), gammas[i].reshape(1, D),
            betas[i].reshape(1, D), ws[i],
        )

    s = _edge_call(epk, y)
    emb, logits, probs = _tc_last(
        s[0], s[1], y, dinv,
        biases[3].reshape(1, D), gammas[3].reshape(1, D), betas[3].reshape(1, D),
        batch.reshape(1, N),
        Wp1, bp1.reshape(1, -1), Wp2, bp2.reshape(1, -1), Wp3, bp3.reshape(1, -1),
    )
    return emb, logits, probs


# speedup vs baseline: 1.0262x; 1.0139x over previous
import functools

import jax
import jax.numpy as jnp
from jax import lax
from jax.experimental import pallas as pl
from jax.experimental.pallas import tpu as pltpu
from jax.experimental.pallas import tpu_sc as plsc

N = 10000
E = 320000
D = 128
NG = 16

NC = 2
NS = 16
NW = NC * NS
EPW = E // NW
CHUNK = 80
NCH = EPW // CHUNK
NPAD = 10240
RPS = NPAD // NS
ZR = 16
DPW = NPAD // NS


def _sc_mesh():
    return plsc.VectorSubcoreMesh(core_axis_name="c", subcore_axis_name="s")


def _deg_body(epk_hbm, out_hbm, pidx_v, didx_v, ones_v, zb_v, acc, dsem0, dsem1):
    c = lax.axis_index("c")
    s = lax.axis_index("s")
    wid = s * NC + c

    z16 = jnp.zeros((16,), jnp.float32)
    o16 = jnp.ones((16,), jnp.float32)

    def fill(i, _):
        zb_v[pl.ds(i * 16, 16)] = z16
        return 0

    lax.fori_loop(0, DPW // 16, fill, 0, unroll=8)
    for i in range(CHUNK // 16):
        ones_v[pl.ds(i * 16, 16)] = o16
    pltpu.sync_copy(zb_v, acc.at[pl.ds(s * DPW, DPW)])
    pltpu.sync_copy(epk_hbm.at[wid], pidx_v)
    plsc.subcore_barrier()

    def unpack(k, buf):
        for i in range(CHUNK // 16):
            p = pidx_v[k, pl.ds(i * 16, 16)]
            didx_v[buf, pl.ds(i * 16, 16)] = lax.shift_right_logical(p, 16)

    def s_start(buf, sem):
        pltpu.make_async_copy(ones_v, acc.at[didx_v.at[buf]], sem).start(add=True)

    def s_wait(buf, sem):
        pltpu.make_async_copy(ones_v, acc.at[didx_v.at[buf]], sem).wait()

    unpack(0, 0)
    s_start(0, dsem0)

    def pair(kk, _):
        k = kk * 2
        unpack(k + 1, 1)
        s_start(1, dsem1)
        s_wait(0, dsem0)
        unpack(k + 2, 0)
        s_start(0, dsem0)
        s_wait(1, dsem1)
        return 0

    lax.fori_loop(0, (NCH - 1) // 2, pair, 0)
    s_wait(0, dsem0)
    plsc.subcore_barrier()
    pltpu.sync_copy(acc.at[pl.ds(s * DPW, DPW)], out_hbm.at[c, pl.ds(s * DPW, DPW)])


def _deg_call(epk):
    return pl.kernel(
        _deg_body,
        out_type=jax.ShapeDtypeStruct((NC, NPAD), jnp.float32),
        mesh=_sc_mesh(),
        scratch_types=[
            pltpu.VMEM((NCH, CHUNK), jnp.int32),
            pltpu.VMEM((2, CHUNK), jnp.int32),
            pltpu.VMEM((CHUNK,), jnp.float32),
            pltpu.VMEM((DPW,), jnp.float32),
            pltpu.VMEM_SHARED((NPAD,), jnp.float32),
            pltpu.SemaphoreType.DMA,
            pltpu.SemaphoreType.DMA,
        ],
    )(epk)


def _edge_body(epk_hbm, y_hbm, out_hbm, pidx_v, sidx_v, didx_v, rows_v, zb_v, acc,
               sem0, sem1, isem):
    c = lax.axis_index("c")
    s = lax.axis_index("s")
    wid = s * NC + c

    icp = pltpu.make_async_copy(epk_hbm.at[wid], pidx_v, isem)
    icp.start()

    z16 = jnp.zeros((16,), jnp.float32)

    def fill(i, _):
        zb_v[i // 8, pl.ds((i % 8) * 16, 16)] = z16
        return 0

    lax.fori_loop(0, ZR * (D // 16), fill, 0, unroll=8)

    def zero(j, _):
        pltpu.sync_copy(zb_v, acc.at[pl.ds(s * RPS + j * ZR, ZR), :])
        return 0

    lax.fori_loop(0, RPS // ZR, zero, 0)
    icp.wait()
    plsc.subcore_barrier()

    def unpack(k, buf):
        for i in range(CHUNK // 16):
            p = pidx_v[k, pl.ds(i * 16, 16)]
            sidx_v[buf, pl.ds(i * 16, 16)] = lax.bitwise_and(p, 0xFFFF)
            didx_v[buf, pl.ds(i * 16, 16)] = lax.shift_right_logical(p, 16)

    def g_start(buf, sem):
        pltpu.make_async_copy(y_hbm.at[sidx_v.at[buf]], rows_v.at[buf], sem).start()

    def g_wait(buf, sem):
        pltpu.make_async_copy(y_hbm.at[sidx_v.at[buf]], rows_v.at[buf], sem).wait()

    def sc_add(buf):
        pltpu.sync_copy(rows_v.at[buf], acc.at[didx_v.at[buf]], add=True)

    unpack(0, 0)
    g_start(0, sem0)

    def pair(kk, _):
        k = kk * 2
        unpack(k + 1, 1)
        g_start(1, sem1)
        g_wait(0, sem0)
        sc_add(0)
        unpack(k + 2, 0)
        g_start(0, sem0)
        g_wait(1, sem1)
        sc_add(1)
        return 0

    lax.fori_loop(0, (NCH - 1) // 2, pair, 0)
    g_wait(0, sem0)
    sc_add(0)

    plsc.subcore_barrier()
    pltpu.sync_copy(
        acc.at[pl.ds(s * RPS, RPS), :], out_hbm.at[c, pl.ds(s * RPS, RPS), :]
    )


def _edge_call(epk, y):
    return pl.kernel(
        _edge_body,
        out_type=jax.ShapeDtypeStruct((NC, NPAD, D), jnp.float32),
        mesh=_sc_mesh(),
        scratch_types=[
            pltpu.VMEM((NCH, CHUNK), jnp.int32),
            pltpu.VMEM((2, CHUNK), jnp.int32),
            pltpu.VMEM((2, CHUNK), jnp.int32),
            pltpu.VMEM((2, CHUNK, D), jnp.float32),
            pltpu.VMEM((ZR, D), jnp.float32),
            pltpu.VMEM_SHARED((NPAD, D), jnp.float32),
            pltpu.SemaphoreType.DMA,
            pltpu.SemaphoreType.DMA,
            pltpu.SemaphoreType.DMA,
        ],
    )(epk, y)


_BR = 1000
_GRID = N // _BR


def _tc_first_body(x_ref, w_ref, d0_ref, d1_ref, y_ref, dinv_ref):
    deg = d0_ref[...] + d1_ref[...] + 1.0
    dinv = lax.rsqrt(deg)
    xw = jnp.dot(x_ref[...], w_ref[...], preferred_element_type=jnp.float32)
    y_ref[...] = xw * dinv
    dinv_ref[...] = dinv


def _tc_first(x, w1, deg0, deg1):
    return pl.pallas_call(
        _tc_first_body,
        grid=(_GRID,),
        in_specs=[
            pl.BlockSpec((_BR, D), lambda i: (i, 0)),
            pl.BlockSpec((D, D), lambda i: (0, 0)),
            pl.BlockSpec((_BR, 1), lambda i: (i, 0)),
            pl.BlockSpec((_BR, 1), lambda i: (i, 0)),
        ],
        out_specs=[
            pl.BlockSpec((_BR, D), lambda i: (i, 0)),
            pl.BlockSpec((_BR, 1), lambda i: (i, 0)),
        ],
        out_shape=[
            jax.ShapeDtypeStruct((N, D), jnp.float32),
            jax.ShapeDtypeStruct((N, 1), jnp.float32),
        ],
    )(x, w1, deg0, deg1)


def _ln(h, g, b, eps=1e-5):
    mu = jnp.mean(h, axis=-1, keepdims=True)
    var = jnp.mean((h - mu) ** 2, axis=-1, keepdims=True)
    return (h - mu) * lax.rsqrt(var + eps) * g + b


def _tc_mid_body(s0_ref, s1_ref, y_ref, dinv_ref, b_ref, g_ref, bl_ref, w_ref, o_ref):
    dinv = dinv_ref[...]
    conv = dinv * (s0_ref[...] + s1_ref[...] + y_ref[...]) + b_ref[...]
    h = _ln(jnp.maximum(conv, 0.0), g_ref[...], bl_ref[...])
    xw = jnp.dot(h, w_ref[...], preferred_element_type=jnp.float32)
    o_ref[...] = xw * dinv


def _tc_mid(s0, s1, y, dinv, b, g, bl, w_next):
    row = lambda i: (i, 0)
    return pl.pallas_call(
        _tc_mid_body,
        grid=(_GRID,),
        in_specs=[
            pl.BlockSpec((_BR, D), row),
            pl.BlockSpec((_BR, D), row),
            pl.BlockSpec((_BR, D), row),
            pl.BlockSpec((_BR, 1), row),
            pl.BlockSpec((1, D), lambda i: (0, 0)),
            pl.BlockSpec((1, D), lambda i: (0, 0)),
            pl.BlockSpec((1, D), lambda i: (0, 0)),
            pl.BlockSpec((D, D), lambda i: (0, 0)),
        ],
        out_specs=pl.BlockSpec((_BR, D), row),
        out_shape=jax.ShapeDtypeStruct((N, D), jnp.float32),
    )(s0, s1, y, dinv, b, g, bl, w_next)


def _tc_last_body(
    s0_ref, s1_ref, y_ref, dinv_ref, b_ref, g_ref, bl_ref, batch_ref,
    wp1_ref, bp1_ref, wp2_ref, bp2_ref, wp3_ref, bp3_ref,
    emb_ref, logits_ref, probs_ref,
):
    dinv = dinv_ref[...]
    emb = dinv * (s0_ref[...] + s1_ref[...] + y_ref[...]) + b_ref[...]
    emb_ref[...] = emb
    h = _ln(jnp.maximum(emb, 0.0), g_ref[...], bl_ref[...])

    gids = lax.broadcasted_iota(jnp.int32, (NG, N), 0)
    onehot = (gids == batch_ref[...]).astype(jnp.float32)
    sums = jnp.dot(onehot, h, preferred_element_type=jnp.float32)
    cnt = jnp.sum(onehot, axis=1, keepdims=True)
    pooled = sums / jnp.maximum(cnt, 1.0)

    z = jnp.dot(pooled, wp1_ref[...], preferred_element_type=jnp.float32) + bp1_ref[...]
    z = jnp.dot(z, wp2_ref[...], preferred_element_type=jnp.float32) + bp2_ref[...]
    logits = jnp.dot(z, wp3_ref[...], preferred_element_type=jnp.float32) + bp3_ref[...]
    logits_ref[...] = logits
    m = jnp.max(logits, axis=1, keepdims=True)
    lse = jnp.log(jnp.sum(jnp.exp(logits - m), axis=1, keepdims=True)) + m
    probs_ref[...] = logits - lse


def _tc_last(s0, s1, y, dinv, b, g, bl, batch_t, wp1, bp1, wp2, bp2, wp3, bp3):
    z = lambda i: (0, 0)
    return pl.pallas_call(
        _tc_last_body,
        grid=(1,),
        in_specs=[
            pl.BlockSpec((N, D), z),
            pl.BlockSpec((N, D), z),
            pl.BlockSpec((N, D), z),
            pl.BlockSpec((N, 1), z),
            pl.BlockSpec((1, D), z),
            pl.BlockSpec((1, D), z),
            pl.BlockSpec((1, D), z),
            pl.BlockSpec((1, N), z),
            pl.BlockSpec((D, 2 * D), z),
            pl.BlockSpec((1, 2 * D), z),
            pl.BlockSpec((2 * D, D), z),
            pl.BlockSpec((1, D), z),
            pl.BlockSpec((D, NG), z),
            pl.BlockSpec((1, NG), z),
        ],
        out_specs=[
            pl.BlockSpec((N, D), z),
            pl.BlockSpec((NG, NG), z),
            pl.BlockSpec((NG, NG), z),
        ],
        out_shape=[
            jax.ShapeDtypeStruct((N, D), jnp.float32),
            jax.ShapeDtypeStruct((NG, NG), jnp.float32),
            jax.ShapeDtypeStruct((NG, NG), jnp.float32),
        ],
    )(s0, s1, y, dinv, b, g, bl, batch_t, wp1, bp1, wp2, bp2, wp3, bp3)


def kernel(x, edge_index, batch, W1, b1, ln1_g, ln1_b, convs_W, convs_b,
           lns_g, lns_b, Wp1, bp1, Wp2, bp2, Wp3, bp3):
    epk = (edge_index[0] | (edge_index[1] << 16)).reshape(NW, NCH, CHUNK)

    deg = _deg_call(epk)
    deg0 = deg[0, :N].reshape(N, 1)
    deg1 = deg[1, :N].reshape(N, 1)

    y, dinv = _tc_first(x, W1, deg0, deg1)

    biases = [b1] + list(convs_b)
    gammas = [ln1_g] + list(lns_g)
    betas = [ln1_b] + list(lns_b)
    ws = list(convs_W)

    for i in range(3):
        s = _edge_call(epk, y)
        y = _tc_mid(
            s[0], s[1], y, dinv,
            biases[i].reshape(1, D), gammas[i].reshape(1, D),
            betas[i].reshape(1, D), ws[i],
        )

    s = _edge_call(epk, y)
    emb, logits, probs = _tc_last(
        s[0], s[1], y, dinv,
        biases[3].reshape(1, D), gammas[3].reshape(1, D), betas[3].reshape(1, D),
        batch.reshape(1, N),
        Wp1, bp1.reshape(1, -1), Wp2, bp2.reshape(1, -1), Wp3, bp3.reshape(1, -1),
    )
    return emb, logits, probs
